# Initial kernel scaffold; baseline (speedup 1.0000x reference)
import jax
import jax.numpy as jnp
from jax.experimental import pallas as pl


def kernel(features, rbf_expansion, neighbor_list, neighbor_mask, W1, b1, W2, b2):
    raise NotImplementedError("write your pallas kernel here")


# speedup vs baseline: 2.0242x; 2.0242x over previous
import functools

import jax
import jax.numpy as jnp
import numpy as np
from jax import lax
from jax.experimental import pallas as pl
from jax.experimental.pallas import tpu as pltpu
from jax.experimental.pallas import tpu_sc as plsc

LOG2 = float(np.log(2.0))

_NC = 2
_NS = 16
_NW = _NC * _NS


def _sc_gather(table, idx3):
    V, D = table.shape
    nw, n_chunks, ch = idx3.shape
    assert nw == _NW
    b_per_w = n_chunks * ch
    B = nw * b_per_w
    mesh = plsc.VectorSubcoreMesh(
        core_axis_name="c", subcore_axis_name="s", num_cores=_NC, num_subcores=_NS
    )

    @functools.partial(
        pl.kernel,
        mesh=mesh,
        out_type=jax.ShapeDtypeStruct((B, D), jnp.float32),
        scratch_types=[
            pltpu.VMEM((n_chunks, ch), jnp.int32),
            pltpu.VMEM((ch, D), jnp.float32),
            pltpu.SemaphoreType.DMA,
        ],
    )
    def k(table_hbm, idx_hbm, out_hbm, idx_v, rows_v, sem):
        wid = lax.axis_index("s") * _NC + lax.axis_index("c")
        base = wid * b_per_w
        pltpu.sync_copy(idx_hbm.at[wid], idx_v)

        def body(i, carry):
            pltpu.async_copy(table_hbm.at[idx_v.at[i]], rows_v, sem).wait()
            pltpu.sync_copy(rows_v, out_hbm.at[pl.ds(base + i * ch, ch)])
            return carry

        lax.fori_loop(0, n_chunks, body, 0)

    return k(table, idx3)


def _tc_fused(rbf, gathered, mask, W1, b1, W2, b2, tb):
    Bd, N, G = rbf.shape
    F = W2.shape[1]
    grid = (Bd // tb,)

    def body(rbf_ref, g_ref, m_ref, w1_ref, b1_ref, w2_ref, b2_ref, out_ref):
        x = rbf_ref[...].reshape(tb * N, G)
        h = jnp.dot(x, w1_ref[...], preferred_element_type=jnp.float32) + b1_ref[...]
        h = jax.nn.softplus(h) - LOG2
        filt = jnp.dot(h, w2_ref[...], preferred_element_type=jnp.float32) + b2_ref[...]
        g = g_ref[...].reshape(tb * N, F)
        prod = filt * g * m_ref[...]
        out_ref[...] = prod.reshape(tb, N, F).sum(axis=1)

    return pl.pallas_call(
        body,
        grid=grid,
        in_specs=[
            pl.BlockSpec((tb, N, G), lambda i: (i, 0, 0)),
            pl.BlockSpec((tb, N, F), lambda i: (i, 0, 0)),
            pl.BlockSpec((tb * N, 1), lambda i: (i, 0)),
            pl.BlockSpec((G, F), lambda i: (0, 0)),
            pl.BlockSpec((1, F), lambda i: (0, 0)),
            pl.BlockSpec((F, F), lambda i: (0, 0)),
            pl.BlockSpec((1, F), lambda i: (0, 0)),
        ],
        out_specs=pl.BlockSpec((tb, F), lambda i: (i, 0)),
        out_shape=jax.ShapeDtypeStruct((Bd, F), jnp.float32),
    )(rbf, gathered, mask, W1, b1, W2, b2)


def kernel(features, rbf_expansion, neighbor_list, neighbor_mask, W1, b1, W2, b2):
    n_frames, n_beads, n_filters = features.shape
    _, _, n_neighbors = neighbor_list.shape
    assert n_frames == 1
    B = n_beads * n_neighbors

    ch = 80
    b_per_w = B // _NW
    n_chunks = b_per_w // ch
    assert _NW * n_chunks * ch == B

    table = features[0]
    idx3 = neighbor_list[0].reshape(_NW, n_chunks, ch).astype(jnp.int32)
    gathered = _sc_gather(table, idx3)

    out = _tc_fused(
        rbf_expansion[0],
        gathered.reshape(n_beads, n_neighbors, n_filters),
        neighbor_mask[0].reshape(B, 1),
        W1,
        b1.reshape(1, n_filters),
        W2,
        b2.reshape(1, n_filters),
        tb=200,
    )
    return out[None]


# speedup vs baseline: 2.6268x; 1.2977x over previous
import functools

import jax
import jax.numpy as jnp
import numpy as np
from jax import lax
from jax.experimental import pallas as pl
from jax.experimental.pallas import tpu as pltpu
from jax.experimental.pallas import tpu_sc as plsc

LOG2 = float(np.log(2.0))

_NC = 2
_NS = 16
_NW = _NC * _NS


def _sc_gather(table, idx2, ch, dtype):
    V, D = table.shape
    nw, b_per_w = idx2.shape
    assert nw == _NW and b_per_w % ch == 0
    n_chunks = b_per_w // ch
    B = nw * b_per_w
    mesh = plsc.VectorSubcoreMesh(
        core_axis_name="c", subcore_axis_name="s", num_cores=_NC, num_subcores=_NS
    )
    n_stagers = 10
    v_per_s = V // n_stagers
    assert v_per_s % 8 == 0 and n_stagers * v_per_s == V

    @functools.partial(
        pl.kernel,
        mesh=mesh,
        out_type=jax.ShapeDtypeStruct((B, D), dtype),
        scratch_types=[
            pltpu.VMEM((b_per_w,), jnp.int32),
            pltpu.VMEM((2, ch, D), dtype),
            pltpu.VMEM_SHARED((V, D), dtype),
            pltpu.SemaphoreType.DMA,
        ],
    )
    def k(table_hbm, idx_hbm, out_hbm, idx_v, rows_v, table_sh, gsem):
        cid = lax.axis_index("c")
        sid = lax.axis_index("s")
        wid = sid * _NC + cid
        base = wid * b_per_w

        @pl.when(sid < n_stagers)
        def _():
            pltpu.sync_copy(
                table_hbm.at[pl.ds(sid * v_per_s, v_per_s)],
                table_sh.at[pl.ds(sid * v_per_s, v_per_s)],
            )

        pltpu.sync_copy(idx_hbm.at[wid], idx_v)
        plsc.subcore_barrier()

        pltpu.async_copy(table_sh.at[idx_v.at[pl.ds(0, ch)]], rows_v.at[0], gsem)

        def body(i, carry):
            slot = lax.rem(i, 2)

            @pl.when(i + 1 < n_chunks)
            def _():
                pltpu.async_copy(
                    table_sh.at[idx_v.at[pl.ds((i + 1) * ch, ch)]],
                    rows_v.at[1 - slot],
                    gsem,
                )

            pltpu.make_async_copy(
                table_sh.at[idx_v.at[pl.ds(i * ch, ch)]], rows_v.at[slot], gsem
            ).wait()
            pltpu.sync_copy(rows_v.at[slot], out_hbm.at[pl.ds(base + i * ch, ch)])
            return carry

        lax.fori_loop(0, n_chunks, body, 0)

    return k(table, idx2)


def _tc_fused(rbf, gathered, mask, W1, b1, W2, b2, tb):
    Bd, N, G = rbf.shape
    F = W2.shape[1]
    grid = (Bd // tb,)

    def body(rbf_ref, g_ref, m_ref, w1_ref, b1_ref, w2_ref, b2_ref, out_ref):
        x = rbf_ref[...].reshape(tb * N, G).astype(jnp.bfloat16)
        w1 = w1_ref[...].astype(jnp.bfloat16)
        h = jnp.dot(x, w1, preferred_element_type=jnp.float32) + b1_ref[...]
        h = (jax.nn.softplus(h) - LOG2).astype(jnp.bfloat16)
        w2 = w2_ref[...].astype(jnp.bfloat16)
        filt = jnp.dot(h, w2, preferred_element_type=jnp.float32) + b2_ref[...]
        prod = filt * g_ref[...].astype(jnp.float32) * m_ref[...]
        out_ref[...] = prod.reshape(tb, N, F).sum(axis=1)

    return pl.pallas_call(
        body,
        grid=grid,
        in_specs=[
            pl.BlockSpec((tb, N, G), lambda i: (i, 0, 0)),
            pl.BlockSpec((tb * N, F), lambda i: (i, 0)),
            pl.BlockSpec((tb * N, 1), lambda i: (i, 0)),
            pl.BlockSpec((G, F), lambda i: (0, 0)),
            pl.BlockSpec((1, F), lambda i: (0, 0)),
            pl.BlockSpec((F, F), lambda i: (0, 0)),
            pl.BlockSpec((1, F), lambda i: (0, 0)),
        ],
        out_specs=pl.BlockSpec((tb, F), lambda i: (i, 0)),
        out_shape=jax.ShapeDtypeStruct((Bd, F), jnp.float32),
    )(rbf, gathered, mask, W1, b1, W2, b2)


def kernel(features, rbf_expansion, neighbor_list, neighbor_mask, W1, b1, W2, b2):
    n_frames, n_beads, n_filters = features.shape
    _, _, n_neighbors = neighbor_list.shape
    assert n_frames == 1
    B = n_beads * n_neighbors

    ch = 80
    b_per_w = B // _NW
    n_chunks = b_per_w // ch
    assert _NW * n_chunks * ch == B

    idx2 = neighbor_list[0].reshape(_NW, b_per_w).astype(jnp.int32)
    gathered = _sc_gather(features[0], idx2, ch, jnp.float32)

    out = _tc_fused(
        rbf_expansion[0],
        gathered,
        neighbor_mask[0].reshape(B, 1),
        W1,
        b1.reshape(1, n_filters),
        W2,
        b2.reshape(1, n_filters),
        tb=200,
    )
    return out[None]
